# baseline (speedup 1.0000x reference)
import functools

import jax
import jax.numpy as jnp
from jax import lax
from jax.experimental import pallas as pl
from jax.experimental.pallas import tpu as pltpu
from jax.experimental.pallas import tpu_sc as plsc

EMB_DIM = 128
N_EDGES = 320000
T_ROWS = 64
NUM_CORES = 2
NUM_SUBCORES = 16
NUM_WORKERS = NUM_CORES * NUM_SUBCORES
LANES = 16
CHUNK = 80
NSETS = 9
PREF = 4

BPW = N_EDGES // NUM_WORKERS
NCHUNKS = BPW // CHUNK
NBODY = -(-(NCHUNKS + NSETS - PREF) // NSETS) * NSETS


def _table_body(w0_ref, w1_ref, w2_ref, t_ref):
    t_ref[...] = jnp.zeros((T_ROWS, EMB_DIM), jnp.float32)
    for i in range(5):
        for j in range(6):
            for k in range(2):
                r = (i * 6 + j) * 2 + k
                t_ref[pl.ds(r, 1), :] = (
                    w0_ref[pl.ds(i, 1), :]
                    + w1_ref[pl.ds(j, 1), :]
                    + w2_ref[pl.ds(k, 1), :]
                )


def _build_table(W0, W1, W2):
    return pl.pallas_call(
        _table_body,
        out_shape=jax.ShapeDtypeStruct((T_ROWS, EMB_DIM), jnp.float32),
    )(W0, W1, W2)


def _sc_lookup(a0, a1, a2, table):
    mesh = plsc.VectorSubcoreMesh(core_axis_name="c", subcore_axis_name="s")

    @functools.partial(
        pl.kernel,
        mesh=mesh,
        out_type=jax.ShapeDtypeStruct((N_EDGES, EMB_DIM), jnp.float32),
        scratch_types=[
            pltpu.VMEM((BPW,), jnp.int32),
            pltpu.VMEM((BPW,), jnp.int32),
            pltpu.VMEM((BPW,), jnp.int32),
            pltpu.VMEM_SHARED((T_ROWS, EMB_DIM), jnp.float32),
        ]
        + [pltpu.VMEM((CHUNK, EMB_DIM), jnp.float32) for _ in range(NSETS)]
        + [pltpu.SemaphoreType.DMA for _ in range(2 * NSETS)],
    )
    def body(a0_hbm, a1_hbm, a2_hbm, t_hbm, out_hbm, c0_v, c1_v, c2_v,
             t_sh, *rest):
        bufs = rest[:NSETS]
        gsems = rest[NSETS:2 * NSETS]
        ssems = rest[2 * NSETS:]
        wid = lax.axis_index("s") * NUM_CORES + lax.axis_index("c")
        base = wid * BPW

        @pl.when(lax.axis_index("s") == 0)
        def _():
            pltpu.sync_copy(t_hbm, t_sh)

        cp0 = pltpu.async_copy(a0_hbm.at[pl.ds(base, BPW)], c0_v, gsems[0])
        cp1 = pltpu.async_copy(a1_hbm.at[pl.ds(base, BPW)], c1_v, gsems[1])
        cp2 = pltpu.async_copy(a2_hbm.at[pl.ds(base, BPW)], c2_v, gsems[2])
        cp0.wait()
        cp1.wait()
        cp2.wait()

        def compute_codes(i):
            for u in range(CHUNK // LANES):
                o = pl.multiple_of(i * CHUNK + u * LANES, LANES)
                c0_v[pl.ds(o, LANES)] = (
                    c0_v[pl.ds(o, LANES)] * 12
                    + c1_v[pl.ds(o, LANES)] * 2
                    + c2_v[pl.ds(o, LANES)]
                )

        plsc.subcore_barrier()

        def fire_gather(i, p):
            off = pl.multiple_of(i * CHUNK, CHUNK)
            idx = c0_v.at[pl.ds(off, CHUNK)]
            pltpu.async_copy(t_sh.at[idx], bufs[p], gsems[p])

        def fire_scatter(i, p):
            off = pl.multiple_of(i * CHUNK, CHUNK)
            pltpu.async_copy(bufs[p], out_hbm.at[pl.ds(base + off, CHUNK)],
                             ssems[p])

        def drain_gather(p):
            pltpu.make_async_copy(out_hbm.at[pl.ds(0, CHUNK)], bufs[p],
                                  gsems[p]).wait()

        def drain_scatter(p):
            pltpu.make_async_copy(bufs[p], out_hbm.at[pl.ds(0, CHUNK)],
                                  ssems[p]).wait()

        for c in range(PREF):
            compute_codes(c)
            fire_gather(c, c)

        def super_body(s, _):
            for p in range(NSETS):
                g = s * NSETS + p
                sp = (p + PREF) % NSETS

                @pl.when((g >= NSETS - PREF)
                         & (g < NCHUNKS + NSETS - PREF))
                def _():
                    drain_scatter(sp)

                @pl.when(g + PREF < NCHUNKS)
                def _():
                    compute_codes(g + PREF)
                    fire_gather(g + PREF, sp)

                @pl.when(g < NCHUNKS)
                def _():
                    drain_gather(p)
                    fire_scatter(g, p)

            return 0

        lax.fori_loop(0, NBODY // NSETS, super_body, 0)

    return body(a0, a1, a2, table)


def kernel(edge_attr, W0, W1, W2):
    table = _build_table(W0, W1, W2)
    a0 = edge_attr[:, 0]
    a1 = edge_attr[:, 1]
    a2 = edge_attr[:, 2]
    return _sc_lookup(a0, a1, a2, table)


# speedup vs baseline: 1.0020x; 1.0020x over previous
import functools

import jax
import jax.numpy as jnp
from jax import lax
from jax.experimental import pallas as pl
from jax.experimental.pallas import tpu as pltpu
from jax.experimental.pallas import tpu_sc as plsc

EMB_DIM = 128
N_EDGES = 320000
T_ROWS = 64
NUM_CORES = 2
NUM_SUBCORES = 16
NUM_WORKERS = NUM_CORES * NUM_SUBCORES
LANES = 16
CHUNK = 80
NSETS = 8
PREF = 4

BPW = N_EDGES // NUM_WORKERS
NCHUNKS = BPW // CHUNK
NBODY = -(-(NCHUNKS + PREF) // NSETS) * NSETS


def _table_body(w0_ref, w1_ref, w2_ref, t_ref):
    t_ref[...] = jnp.zeros((T_ROWS, EMB_DIM), jnp.float32)
    for i in range(5):
        for j in range(6):
            for k in range(2):
                r = (i * 6 + j) * 2 + k
                t_ref[pl.ds(r, 1), :] = (
                    w0_ref[pl.ds(i, 1), :]
                    + w1_ref[pl.ds(j, 1), :]
                    + w2_ref[pl.ds(k, 1), :]
                )


def _build_table(W0, W1, W2):
    return pl.pallas_call(
        _table_body,
        out_shape=jax.ShapeDtypeStruct((T_ROWS, EMB_DIM), jnp.float32),
    )(W0, W1, W2)


def _sc_lookup(a0, a1, a2, table):
    mesh = plsc.VectorSubcoreMesh(core_axis_name="c", subcore_axis_name="s")

    @functools.partial(
        pl.kernel,
        mesh=mesh,
        out_type=jax.ShapeDtypeStruct((N_EDGES, EMB_DIM), jnp.float32),
        scratch_types=[
            pltpu.VMEM((BPW,), jnp.int32),
            pltpu.VMEM((BPW,), jnp.int32),
            pltpu.VMEM((BPW,), jnp.int32),
            pltpu.VMEM((BPW,), jnp.int32),
            pltpu.VMEM_SHARED((T_ROWS, EMB_DIM), jnp.float32),
        ]
        + [pltpu.VMEM((CHUNK, EMB_DIM), jnp.float32) for _ in range(NSETS)]
        + [pltpu.SemaphoreType.DMA for _ in range(2 * NSETS)],
    )
    def body(a0_hbm, a1_hbm, a2_hbm, t_hbm, out_hbm, c0_v, c1_v, c2_v,
             codes_v, t_sh, *rest):
        bufs = rest[:NSETS]
        gsems = rest[NSETS:2 * NSETS]
        ssems = rest[2 * NSETS:]
        wid = lax.axis_index("s") * NUM_CORES + lax.axis_index("c")
        base = wid * BPW

        @pl.when(lax.axis_index("s") == 0)
        def _():
            pltpu.sync_copy(t_hbm, t_sh)

        cp0 = pltpu.async_copy(a0_hbm.at[pl.ds(base, BPW)], c0_v, gsems[0])
        cp1 = pltpu.async_copy(a1_hbm.at[pl.ds(base, BPW)], c1_v, gsems[1])
        cp2 = pltpu.async_copy(a2_hbm.at[pl.ds(base, BPW)], c2_v, gsems[2])
        cp0.wait()
        cp1.wait()
        cp2.wait()

        def compute_codes(i):
            for u in range(CHUNK // LANES):
                o = pl.multiple_of(i * CHUNK + u * LANES, LANES)
                codes_v[pl.ds(o, LANES)] = (
                    c0_v[pl.ds(o, LANES)] * 12
                    + c1_v[pl.ds(o, LANES)] * 2
                    + c2_v[pl.ds(o, LANES)]
                )

        plsc.subcore_barrier()

        def fire_gather(i, p):
            off = pl.multiple_of(i * CHUNK, CHUNK)
            idx = codes_v.at[pl.ds(off, CHUNK)]
            pltpu.async_copy(t_sh.at[idx], bufs[p], gsems[p])

        def fire_scatter(i, p):
            off = pl.multiple_of(i * CHUNK, CHUNK)
            pltpu.async_copy(bufs[p], out_hbm.at[pl.ds(base + off, CHUNK)],
                             ssems[p])

        def drain_gather(p):
            pltpu.make_async_copy(out_hbm.at[pl.ds(0, CHUNK)], bufs[p],
                                  gsems[p]).wait()

        def drain_scatter(p):
            pltpu.make_async_copy(bufs[p], out_hbm.at[pl.ds(0, CHUNK)],
                                  ssems[p]).wait()

        for c in range(PREF):
            compute_codes(c)
            fire_gather(c, c)

        def super_body(s, _):
            for p in range(NSETS):
                g = s * NSETS + p
                sp = (p + PREF) % NSETS

                @pl.when((g >= PREF) & (g < NCHUNKS + PREF))
                def _():
                    drain_scatter(sp)

                @pl.when(g + PREF < NCHUNKS)
                def _():
                    compute_codes(g + PREF)
                    fire_gather(g + PREF, sp)

                @pl.when(g < NCHUNKS)
                def _():
                    drain_gather(p)
                    fire_scatter(g, p)

            return 0

        lax.fori_loop(0, NBODY // NSETS, super_body, 0)

    return body(a0, a1, a2, table)


def kernel(edge_attr, W0, W1, W2):
    table = _build_table(W0, W1, W2)
    a0 = edge_attr[:, 0]
    a1 = edge_attr[:, 1]
    a2 = edge_attr[:, 2]
    return _sc_lookup(a0, a1, a2, table)
